# baseline (speedup 1.0000x reference)
import functools
import jax
import jax.numpy as jnp
from jax import lax
from jax.experimental import pallas as pl
from jax.experimental.pallas import tpu as pltpu
from jax.experimental.pallas import tpu_sc as plsc

_N = 10000
_E = 320000
_D = 128
_ED = 16

_NW = 32
_EPW = _E // _NW
_C = 80
_NCHUNK = _EPW // _C
_NT = 10240
_STRIPE = _NT // 16
_BB = 32


def _zero_vmem_2d(ref, rows, cols):
    per_row = cols // 16
    zeros16 = jnp.zeros((16,), jnp.float32)

    def body(t, carry):
        r = t // per_row
        c = (t % per_row) * 16
        ref[r, pl.ds(c, 16)] = zeros16
        return carry

    lax.fori_loop(0, rows * per_row, body, 0)


def _sc_seg(rows, ei1):
    mesh = plsc.VectorSubcoreMesh(core_axis_name="c", subcore_axis_name="s")

    @functools.partial(
        pl.kernel,
        mesh=mesh,
        out_type=jax.ShapeDtypeStruct((2, _NT, _D), jnp.float32),
        scratch_types=[
            pltpu.VMEM((_C,), jnp.int32),
            pltpu.VMEM((_C,), jnp.int32),
            pltpu.VMEM((_C,), jnp.int32),
            pltpu.VMEM((_C,), jnp.int32),
            pltpu.VMEM((_C, _D), jnp.float32),
            pltpu.VMEM((_C, _D), jnp.float32),
            pltpu.VMEM((_BB, _D), jnp.float32),
            pltpu.VMEM_SHARED((_NT, _D), jnp.float32),
            pltpu.SemaphoreType.DMA,
            pltpu.SemaphoreType.DMA,
            pltpu.SemaphoreType.DMA,
            pltpu.SemaphoreType.DMA,
            pltpu.SemaphoreType.DMA,
            pltpu.SemaphoreType.DMA,
        ],
    )
    def k(rows_hbm, ei_hbm, out_hbm, sb0_v, sb1_v, db0_v, db1_v,
          r0_v, r1_v, bounce_v, table_sh,
          g0_sem, g1_sem, s0_sem, s1_sem, d0_sem, d1_sem):
        c = lax.axis_index("c")
        s = lax.axis_index("s")
        wid = s * 2 + c

        sbufs = (sb0_v, sb1_v)
        dbufs = (db0_v, db1_v)
        rbufs = (r0_v, r1_v)
        gsems = (g0_sem, g1_sem)
        ssems = (s0_sem, s1_sem)
        dsems = (d0_sem, d1_sem)

        def sidx_load(j, b):
            pltpu.async_copy(ei_hbm.at[pl.ds(wid * _EPW + j * _C, _C)],
                             sbufs[b], ssems[b])

        def sidx_wait(j, b):
            pltpu.make_async_copy(ei_hbm.at[pl.ds(wid * _EPW + j * _C, _C)],
                                  sbufs[b], ssems[b]).wait()

        def didx_load(j, b):
            pltpu.async_copy(ei_hbm.at[pl.ds(_E + wid * _EPW + j * _C, _C)],
                             dbufs[b], dsems[b])

        def didx_wait(j, b):
            pltpu.make_async_copy(
                ei_hbm.at[pl.ds(_E + wid * _EPW + j * _C, _C)],
                dbufs[b], dsems[b]).wait()

        def gather(j, b):
            pltpu.async_copy(rows_hbm.at[sbufs[b]], rbufs[b], gsems[b])

        def gwait(j, b):
            pltpu.make_async_copy(rows_hbm.at[sbufs[b]], rbufs[b],
                                  gsems[b]).wait()

        def scatter(j, b):
            pltpu.sync_copy(rbufs[b], table_sh.at[dbufs[b]], add=True)

        sidx_load(0, 0)
        didx_load(0, 0)

        _zero_vmem_2d(bounce_v, _BB, _D)

        def zbody(j, carry):
            pltpu.sync_copy(bounce_v,
                            table_sh.at[pl.ds(s * _STRIPE + j * _BB, _BB)])
            return carry

        lax.fori_loop(0, _STRIPE // _BB, zbody, 0)
        sidx_wait(0, 0)
        plsc.subcore_barrier()

        gather(0, 0)
        sidx_load(1, 1)
        didx_load(1, 1)

        def body(g, carry):
            j0 = g * 2
            j1 = j0 + 1

            @pl.when(j1 < _NCHUNK)
            def _():
                sidx_wait(j1, 1)
                gather(j1, 1)

            gwait(j0, 0)
            didx_wait(j0, 0)
            scatter(j0, 0)

            @pl.when(j0 + 2 < _NCHUNK)
            def _():
                sidx_load(j0 + 2, 0)
                didx_load(j0 + 2, 0)

            @pl.when(j1 < _NCHUNK)
            def _():
                @pl.when(j1 + 1 < _NCHUNK)
                def _():
                    sidx_wait(j1 + 1, 0)
                    gather(j1 + 1, 0)

                gwait(j1, 1)
                didx_wait(j1, 1)
                scatter(j1, 1)

                @pl.when(j1 + 2 < _NCHUNK)
                def _():
                    sidx_load(j1 + 2, 1)
                    didx_load(j1 + 2, 1)

            return carry

        lax.fori_loop(0, (_NCHUNK + 1) // 2, body, 0)
        plsc.subcore_barrier()

        def rbody(j, carry):
            r0 = s * _STRIPE + j * _BB
            pltpu.sync_copy(table_sh.at[pl.ds(r0, _BB)], bounce_v)
            pltpu.sync_copy(bounce_v, out_hbm.at[c, pl.ds(r0, _BB)])
            return carry

        lax.fori_loop(0, _STRIPE // _BB, rbody, 0)

    return k(rows, ei1)


def _sc_eac(ea_aug, ei1):
    mesh = plsc.VectorSubcoreMesh(core_axis_name="c", subcore_axis_name="s")

    @functools.partial(
        pl.kernel,
        mesh=mesh,
        out_type=jax.ShapeDtypeStruct((2, _NT, _D), jnp.float32),
        scratch_types=[
            pltpu.VMEM((_C,), jnp.int32),
            pltpu.VMEM((_C,), jnp.int32),
            pltpu.VMEM((_C, _D), jnp.float32),
            pltpu.VMEM((_C, _D), jnp.float32),
            pltpu.VMEM((_BB, _D), jnp.float32),
            pltpu.VMEM_SHARED((_NT, _D), jnp.float32),
            pltpu.SemaphoreType.DMA,
            pltpu.SemaphoreType.DMA,
            pltpu.SemaphoreType.DMA,
            pltpu.SemaphoreType.DMA,
        ],
    )
    def k(ea_hbm, ei_hbm, out_hbm, db0_v, db1_v, r0_v, r1_v, bounce_v,
          table_sh, g0_sem, g1_sem, d0_sem, d1_sem):
        c = lax.axis_index("c")
        s = lax.axis_index("s")
        wid = s * 2 + c

        dbufs = (db0_v, db1_v)
        rbufs = (r0_v, r1_v)
        gsems = (g0_sem, g1_sem)
        dsems = (d0_sem, d1_sem)

        def didx_load(j, b):
            pltpu.async_copy(ei_hbm.at[pl.ds(_E + wid * _EPW + j * _C, _C)],
                             dbufs[b], dsems[b])

        def didx_wait(j, b):
            pltpu.make_async_copy(
                ei_hbm.at[pl.ds(_E + wid * _EPW + j * _C, _C)],
                dbufs[b], dsems[b]).wait()

        def rload(j, b):
            pltpu.async_copy(ea_hbm.at[pl.ds(wid * _EPW + j * _C, _C)],
                             rbufs[b], gsems[b])

        def rwait(j, b):
            pltpu.make_async_copy(ea_hbm.at[pl.ds(wid * _EPW + j * _C, _C)],
                                  rbufs[b], gsems[b]).wait()

        def scatter(j, b):
            pltpu.sync_copy(rbufs[b], table_sh.at[dbufs[b]], add=True)

        rload(0, 0)
        didx_load(0, 0)

        _zero_vmem_2d(bounce_v, _BB, _D)

        def zbody(j, carry):
            pltpu.sync_copy(bounce_v,
                            table_sh.at[pl.ds(s * _STRIPE + j * _BB, _BB)])
            return carry

        lax.fori_loop(0, _STRIPE // _BB, zbody, 0)
        plsc.subcore_barrier()

        rload(1, 1)
        didx_load(1, 1)

        def body(g, carry):
            j0 = g * 2
            j1 = j0 + 1

            rwait(j0, 0)
            didx_wait(j0, 0)
            scatter(j0, 0)

            @pl.when(j0 + 2 < _NCHUNK)
            def _():
                rload(j0 + 2, 0)
                didx_load(j0 + 2, 0)

            @pl.when(j1 < _NCHUNK)
            def _():
                rwait(j1, 1)
                didx_wait(j1, 1)
                scatter(j1, 1)

                @pl.when(j1 + 2 < _NCHUNK)
                def _():
                    rload(j1 + 2, 1)
                    didx_load(j1 + 2, 1)

            return carry

        lax.fori_loop(0, (_NCHUNK + 1) // 2, body, 0)
        plsc.subcore_barrier()

        def rbody(j, carry):
            r0 = s * _STRIPE + j * _BB
            pltpu.sync_copy(table_sh.at[pl.ds(r0, _BB)], bounce_v)
            pltpu.sync_copy(bounce_v, out_hbm.at[c, pl.ds(r0, _BB)])
            return carry

        lax.fori_loop(0, _STRIPE // _BB, rbody, 0)

    return k(ea_aug, ei1)


def _dot(a, b):
    return jnp.dot(a, b, preferred_element_type=jnp.float32)


def _tc_ea(ea_packed, w_bd, b_tile):
    def body(x_ref, w_ref, b_ref, o_ref):
        o_ref[...] = jnp.maximum(_dot(x_ref[...], w_ref[...]) + b_ref[...], 0.0)

    nblk = 40
    return pl.pallas_call(
        body,
        grid=(nblk,),
        in_specs=[
            pl.BlockSpec((_E // 8 // nblk, 128), lambda i: (i, 0)),
            pl.BlockSpec((128, 128), lambda i: (0, 0)),
            pl.BlockSpec((1, 128), lambda i: (0, 0)),
        ],
        out_specs=pl.BlockSpec((_E // 8 // nblk, 128), lambda i: (i, 0)),
        out_shape=jax.ShapeDtypeStruct((_E // 8, 128), jnp.float32),
    )(ea_packed, w_bd, b_tile)


def _tc_ea_aug(edge_attr, w_aug, b_aug):
    def body(x_ref, w_ref, b_ref, o_ref):
        o_ref[...] = jnp.maximum(_dot(x_ref[...], w_ref[...]) + b_ref[...], 0.0)

    nblk = 40
    blk = _E // nblk
    return pl.pallas_call(
        body,
        grid=(nblk,),
        in_specs=[
            pl.BlockSpec((blk, _ED), lambda i: (i, 0)),
            pl.BlockSpec((_ED, _D), lambda i: (0, 0)),
            pl.BlockSpec((1, _D), lambda i: (0, 0)),
        ],
        out_specs=pl.BlockSpec((blk, _D), lambda i: (i, 0)),
        out_shape=jax.ShapeDtypeStruct((_E, _D), jnp.float32),
    )(edge_attr, w_aug, b_aug)


def _tc_head(x, enc_w, enc_b, node_w, node_b):
    def body(x_ref, we_ref, be_ref, wn_ref, bn_ref, o_ref):
        h0 = jnp.maximum(_dot(x_ref[...], we_ref[...]) + be_ref[...], 0.0)
        o_ref[...] = _dot(h0, wn_ref[...]) + bn_ref[...]

    blk = 400
    return pl.pallas_call(
        body,
        grid=(_N // blk,),
        in_specs=[
            pl.BlockSpec((blk, _D), lambda i: (i, 0)),
            pl.BlockSpec((_D, _D), lambda i: (0, 0)),
            pl.BlockSpec((1, _D), lambda i: (0, 0)),
            pl.BlockSpec((_D, _D), lambda i: (0, 0)),
            pl.BlockSpec((1, _D), lambda i: (0, 0)),
        ],
        out_specs=pl.BlockSpec((blk, _D), lambda i: (i, 0)),
        out_shape=jax.ShapeDtypeStruct((_N, _D), jnp.float32),
    )(x, enc_w, enc_b, node_w, node_b)


def _tc_post(xh, seg, eac, wt2, m2aug, upd_top, upd_b, ln_g, ln_b,
             next_w=None, next_b=None):
    last = next_w is None

    def body(*refs):
        if last:
            (xh_ref, s0_ref, s1_ref, ec0_ref, ec1_ref,
             wt2_ref, m2_ref, ut_ref, ub_ref, g_ref, b_ref,
             o_ref, hs_ref) = refs
        else:
            (xh_ref, s0_ref, s1_ref, ec0_ref, ec1_ref,
             wt2_ref, m2_ref, ut_ref, ub_ref, g_ref, b_ref,
             nw_ref, nb_ref, o_ref) = refs
        seg_b = s0_ref[...] + s1_ref[...]
        eac_b = ec0_ref[...] + ec1_ref[...]
        cnt = eac_b[:, 16:17]
        pre = _dot(seg_b, wt2_ref[...]) + _dot(eac_b, m2_ref[...])
        inv = 1.0 / jnp.maximum(cnt, 1.0)
        out = _dot(xh_ref[...], ut_ref[...]) + pre * inv + ub_ref[...]
        mu = jnp.mean(out, axis=-1, keepdims=True)
        var = jnp.mean((out - mu) ** 2, axis=-1, keepdims=True)
        out = (out - mu) * lax.rsqrt(var + 1e-5) * g_ref[...] + b_ref[...]
        h = jnp.maximum(out, 0.0)
        if last:
            o_ref[...] = h
            i = pl.program_id(0)

            @pl.when(i == 0)
            def _():
                hs_ref[...] = jnp.zeros_like(hs_ref)

            hs_ref[...] += jnp.sum(h, axis=0, keepdims=True)
        else:
            o_ref[...] = _dot(h, nw_ref[...]) + nb_ref[...]

    blk = 400
    nb = pl.BlockSpec((blk, _D), lambda i: (i, 0))
    cst = lambda r, c: pl.BlockSpec((r, c), lambda i: (0, 0))
    in_specs = [nb, nb, nb, nb, nb,
                cst(_D, _D), cst(_D, _D),
                cst(_D, _D), cst(1, _D), cst(1, _D), cst(1, _D)]
    args = [xh, seg[0], seg[1], eac[0], eac[1],
            wt2, m2aug, upd_top, upd_b, ln_g, ln_b]
    if last:
        out_specs = [nb, pl.BlockSpec((1, _D), lambda i: (0, 0))]
        out_shape = [jax.ShapeDtypeStruct((_N, _D), jnp.float32),
                     jax.ShapeDtypeStruct((1, _D), jnp.float32)]
    else:
        in_specs += [cst(_D, _D), cst(1, _D)]
        args += [next_w, next_b]
        out_specs = nb
        out_shape = jax.ShapeDtypeStruct((_N, _D), jnp.float32)
    return pl.pallas_call(
        body,
        grid=(_N // blk,),
        in_specs=in_specs,
        out_specs=out_specs,
        out_shape=out_shape,
    )(*args)


def _tc_graph(hsum, pool_w, pool_b):
    def body(hs_ref, w_ref, b_ref, o_ref):
        mean = hs_ref[...] * (1.0 / _N)
        o_ref[...] = jnp.maximum(_dot(mean, w_ref[...]) + b_ref[...], 0.0)

    return pl.pallas_call(
        body,
        grid=(1,),
        in_specs=[
            pl.BlockSpec((1, _D), lambda i: (0, 0)),
            pl.BlockSpec((_D, _D), lambda i: (0, 0)),
            pl.BlockSpec((1, _D), lambda i: (0, 0)),
        ],
        out_specs=pl.BlockSpec((1, _D), lambda i: (0, 0)),
        out_shape=jax.ShapeDtypeStruct((1, _D), jnp.float32),
    )(hsum, pool_w, pool_b)


@jax.jit
def kernel(x, edge_index, edge_attr, params):
    ei1 = edge_index.reshape(2 * _E)

    w_bd = jnp.kron(jnp.eye(8, dtype=jnp.float32), params['enc_e_W'])
    b_tile = jnp.tile(params['enc_e_b'], 8)[None, :]
    w_aug = jnp.concatenate(
        [params['enc_e_W'], jnp.zeros((_ED, _D - _ED), jnp.float32)], axis=1)
    b_aug = jnp.concatenate(
        [params['enc_e_b'], jnp.ones((1,), jnp.float32),
         jnp.zeros((_D - _ED - 1,), jnp.float32)])[None, :]
    lw = []
    for lp in params['layers']:
        wt = lp['msg_W'][:_D]
        wb = lp['msg_W'][_D:]
        m = lp['edge_W'] @ wb
        cvec = lp['edge_b'] @ wb + lp['msg_b']
        upd_top = lp['upd_W'][:_D]
        upd_bot = lp['upd_W'][_D:]
        m2 = m @ upd_bot
        c2 = (cvec @ upd_bot)[None, :]
        m2aug = jnp.concatenate(
            [m2, c2, jnp.zeros((_D - _ED - 1, _D), jnp.float32)], axis=0)
        lw.append({
            'wt2': wt @ upd_bot,
            'm2aug': m2aug,
            'upd_top': upd_top,
            'upd_b': lp['upd_b'][None, :],
            'ln_g': lp['ln_g'][None, :],
            'ln_b': lp['ln_b'][None, :],
            'node_w': lp['node_W'],
            'node_b': lp['node_b'][None, :],
        })

    ea_packed = _tc_ea(edge_attr.reshape(_E // 8, 128), w_bd, b_tile)
    ea = ea_packed.reshape(_E, _ED)

    ea_aug = _tc_ea_aug(edge_attr, w_aug, b_aug)
    eac = _sc_eac(ea_aug, ei1)

    xh = _tc_head(x, params['enc_n_W'], params['enc_n_b'][None, :],
                  lw[0]['node_w'], lw[0]['node_b'])

    seg1 = _sc_seg(xh, ei1)
    xh2 = _tc_post(xh, seg1, eac, lw[0]['wt2'], lw[0]['m2aug'],
                   lw[0]['upd_top'], lw[0]['upd_b'],
                   lw[0]['ln_g'], lw[0]['ln_b'],
                   next_w=lw[1]['node_w'], next_b=lw[1]['node_b'])

    seg2 = _sc_seg(xh2, ei1)
    h, hsum = _tc_post(xh2, seg2, eac, lw[1]['wt2'], lw[1]['m2aug'],
                       lw[1]['upd_top'], lw[1]['upd_b'],
                       lw[1]['ln_g'], lw[1]['ln_b'])

    graph = _tc_graph(hsum, params['pool_W'], params['pool_b'][None, :])
    return (h, ea, graph)


# speedup vs baseline: 1.3951x; 1.3951x over previous
import functools
import jax
import jax.numpy as jnp
from jax import lax
from jax.experimental import pallas as pl
from jax.experimental.pallas import tpu as pltpu
from jax.experimental.pallas import tpu_sc as plsc

_N = 10000
_E = 320000
_D = 128
_ED = 16

_NW = 32
_EPW = _E // _NW
_C = 80
_NCHUNK = _EPW // _C
_NT = 10240
_STRIPE = _NT // 16
_BB = 32


def _zero_vmem_2d(ref, rows, cols):
    per_row = cols // 16
    zeros16 = jnp.zeros((16,), jnp.float32)

    def body(t, carry):
        r = t // per_row
        c = (t % per_row) * 16
        ref[r, pl.ds(c, 16)] = zeros16
        return carry

    lax.fori_loop(0, rows * per_row, body, 0)


def _sc_seg(rows, ei1):
    mesh = plsc.VectorSubcoreMesh(core_axis_name="c", subcore_axis_name="s")

    @functools.partial(
        pl.kernel,
        mesh=mesh,
        out_type=jax.ShapeDtypeStruct((2, _NT, _D), jnp.float32),
        scratch_types=[
            pltpu.VMEM((_C,), jnp.int32),
            pltpu.VMEM((_C,), jnp.int32),
            pltpu.VMEM((_C,), jnp.int32),
            pltpu.VMEM((_C,), jnp.int32),
            pltpu.VMEM((_C, _D), jnp.float32),
            pltpu.VMEM((_C, _D), jnp.float32),
            pltpu.VMEM((_BB, _D), jnp.float32),
            pltpu.VMEM_SHARED((_NT, _D), jnp.float32),
            pltpu.SemaphoreType.DMA,
            pltpu.SemaphoreType.DMA,
            pltpu.SemaphoreType.DMA,
            pltpu.SemaphoreType.DMA,
            pltpu.SemaphoreType.DMA,
            pltpu.SemaphoreType.DMA,
        ],
    )
    def k(rows_hbm, ei_hbm, out_hbm, sb0_v, sb1_v, db0_v, db1_v,
          r0_v, r1_v, bounce_v, table_sh,
          g0_sem, g1_sem, s0_sem, s1_sem, d0_sem, d1_sem):
        c = lax.axis_index("c")
        s = lax.axis_index("s")
        wid = s * 2 + c

        sbufs = (sb0_v, sb1_v)
        dbufs = (db0_v, db1_v)
        rbufs = (r0_v, r1_v)
        gsems = (g0_sem, g1_sem)
        ssems = (s0_sem, s1_sem)
        dsems = (d0_sem, d1_sem)

        def sidx_load(j, b):
            pltpu.async_copy(ei_hbm.at[pl.ds(wid * _EPW + j * _C, _C)],
                             sbufs[b], ssems[b])

        def sidx_wait(j, b):
            pltpu.make_async_copy(ei_hbm.at[pl.ds(wid * _EPW + j * _C, _C)],
                                  sbufs[b], ssems[b]).wait()

        def didx_load(j, b):
            pltpu.async_copy(ei_hbm.at[pl.ds(_E + wid * _EPW + j * _C, _C)],
                             dbufs[b], dsems[b])

        def didx_wait(j, b):
            pltpu.make_async_copy(
                ei_hbm.at[pl.ds(_E + wid * _EPW + j * _C, _C)],
                dbufs[b], dsems[b]).wait()

        def gather(j, b):
            pltpu.async_copy(rows_hbm.at[sbufs[b]], rbufs[b], gsems[b])

        def gwait(j, b):
            pltpu.make_async_copy(rows_hbm.at[sbufs[b]], rbufs[b],
                                  gsems[b]).wait()

        def scatter(j, b):
            pltpu.sync_copy(rbufs[b], table_sh.at[dbufs[b]], add=True)

        sidx_load(0, 0)
        didx_load(0, 0)

        _zero_vmem_2d(bounce_v, _BB, _D)

        def zbody(j, carry):
            pltpu.sync_copy(bounce_v,
                            table_sh.at[pl.ds(s * _STRIPE + j * _BB, _BB)])
            return carry

        lax.fori_loop(0, _STRIPE // _BB, zbody, 0)
        sidx_wait(0, 0)
        plsc.subcore_barrier()

        gather(0, 0)
        sidx_load(1, 1)
        didx_load(1, 1)

        def body(g, carry):
            j0 = g * 2
            j1 = j0 + 1

            @pl.when(j1 < _NCHUNK)
            def _():
                sidx_wait(j1, 1)
                gather(j1, 1)

            gwait(j0, 0)
            didx_wait(j0, 0)
            scatter(j0, 0)

            @pl.when(j0 + 2 < _NCHUNK)
            def _():
                sidx_load(j0 + 2, 0)
                didx_load(j0 + 2, 0)

            @pl.when(j1 < _NCHUNK)
            def _():
                @pl.when(j1 + 1 < _NCHUNK)
                def _():
                    sidx_wait(j1 + 1, 0)
                    gather(j1 + 1, 0)

                gwait(j1, 1)
                didx_wait(j1, 1)
                scatter(j1, 1)

                @pl.when(j1 + 2 < _NCHUNK)
                def _():
                    sidx_load(j1 + 2, 1)
                    didx_load(j1 + 2, 1)

            return carry

        lax.fori_loop(0, (_NCHUNK + 1) // 2, body, 0)
        plsc.subcore_barrier()

        def rbody(j, carry):
            r0 = s * _STRIPE + j * _BB
            pltpu.sync_copy(table_sh.at[pl.ds(r0, _BB)], bounce_v)
            pltpu.sync_copy(bounce_v, out_hbm.at[c, pl.ds(r0, _BB)])
            return carry

        lax.fori_loop(0, _STRIPE // _BB, rbody, 0)

    return k(rows, ei1)


def _sc_eac(ea_packed, ei1):
    mesh = plsc.VectorSubcoreMesh(core_axis_name="c", subcore_axis_name="s")
    CE = 64
    TOTC = _E // CE
    KMAX = (TOTC + _NW - 1) // _NW

    @functools.partial(
        pl.kernel,
        mesh=mesh,
        out_type=jax.ShapeDtypeStruct((2, _NT, _D), jnp.float32),
        scratch_types=[
            pltpu.VMEM((CE,), jnp.int32),
            pltpu.VMEM((CE,), jnp.int32),
            pltpu.VMEM((CE * _ED,), jnp.float32),
            pltpu.VMEM((CE * _ED,), jnp.float32),
            pltpu.VMEM((CE, _D), jnp.float32),
            pltpu.VMEM((_BB, _D), jnp.float32),
            pltpu.VMEM_SHARED((_NT, _D), jnp.float32),
            pltpu.SemaphoreType.DMA,
            pltpu.SemaphoreType.DMA,
            pltpu.SemaphoreType.DMA,
            pltpu.SemaphoreType.DMA,
        ],
    )
    def k(eap_hbm, ei_hbm, out_hbm, db0_v, db1_v, p0_v, p1_v, rows_v,
          bounce_v, table_sh, p0_sem, p1_sem, d0_sem, d1_sem):
        c = lax.axis_index("c")
        s = lax.axis_index("s")
        wid = s * 2 + c

        dbufs = (db0_v, db1_v)
        pbufs = (p0_v, p1_v)
        psems = (p0_sem, p1_sem)
        dsems = (d0_sem, d1_sem)

        def ch(kk):
            return wid + kk * _NW

        def didx_load(kk, b):
            pltpu.async_copy(ei_hbm.at[pl.ds(_E + ch(kk) * CE, CE)],
                             dbufs[b], dsems[b])

        def didx_wait(kk, b):
            pltpu.make_async_copy(ei_hbm.at[pl.ds(_E + ch(kk) * CE, CE)],
                                  dbufs[b], dsems[b]).wait()

        def pload(kk, b):
            pltpu.async_copy(eap_hbm.at[pl.ds(ch(kk) * (CE * _ED), CE * _ED)],
                             pbufs[b], psems[b])

        def pwait(kk, b):
            pltpu.make_async_copy(
                eap_hbm.at[pl.ds(ch(kk) * (CE * _ED), CE * _ED)],
                pbufs[b], psems[b]).wait()

        didx_load(0, 0)
        pload(0, 0)
        didx_load(1, 1)
        pload(1, 1)

        _zero_vmem_2d(rows_v, CE, _D)
        onehot = jnp.where(lax.iota(jnp.int32, 16) == 0, 1.0, 0.0)
        for e in range(CE):
            rows_v[e, pl.ds(16, 16)] = onehot

        _zero_vmem_2d(bounce_v, _BB, _D)

        def zbody(j, carry):
            pltpu.sync_copy(bounce_v,
                            table_sh.at[pl.ds(s * _STRIPE + j * _BB, _BB)])
            return carry

        lax.fori_loop(0, _STRIPE // _BB, zbody, 0)
        plsc.subcore_barrier()

        def process(kk, b):
            pwait(kk, b)
            pbuf = pbufs[b]
            for e in range(CE):
                rows_v[e, pl.ds(0, 16)] = pbuf[pl.ds(e * 16, 16)]
            didx_wait(kk, b)
            pltpu.sync_copy(rows_v, table_sh.at[dbufs[b]], add=True)

            @pl.when(ch(kk + 2) < TOTC)
            def _():
                didx_load(kk + 2, b)
                pload(kk + 2, b)

        def body(g, carry):
            k0 = g * 2

            @pl.when(ch(k0) < TOTC)
            def _():
                process(k0, 0)

            @pl.when(ch(k0 + 1) < TOTC)
            def _():
                process(k0 + 1, 1)

            return carry

        lax.fori_loop(0, (KMAX + 1) // 2, body, 0)
        plsc.subcore_barrier()

        def rbody(j, carry):
            r0 = s * _STRIPE + j * _BB
            pltpu.sync_copy(table_sh.at[pl.ds(r0, _BB)], bounce_v)
            pltpu.sync_copy(bounce_v, out_hbm.at[c, pl.ds(r0, _BB)])
            return carry

        lax.fori_loop(0, _STRIPE // _BB, rbody, 0)

    return k(ea_packed.reshape(_E * _ED), ei1)


def _dot(a, b):
    return jnp.dot(a, b, preferred_element_type=jnp.float32)


def _tc_ea(ea_packed, w_bd, b_tile):
    def body(x_ref, w_ref, b_ref, o_ref):
        o_ref[...] = jnp.maximum(_dot(x_ref[...], w_ref[...]) + b_ref[...], 0.0)

    nblk = 40
    return pl.pallas_call(
        body,
        grid=(nblk,),
        in_specs=[
            pl.BlockSpec((_E // 8 // nblk, 128), lambda i: (i, 0)),
            pl.BlockSpec((128, 128), lambda i: (0, 0)),
            pl.BlockSpec((1, 128), lambda i: (0, 0)),
        ],
        out_specs=pl.BlockSpec((_E // 8 // nblk, 128), lambda i: (i, 0)),
        out_shape=jax.ShapeDtypeStruct((_E // 8, 128), jnp.float32),
    )(ea_packed, w_bd, b_tile)


def _tc_head(x, enc_w, enc_b, node_w, node_b):
    def body(x_ref, we_ref, be_ref, wn_ref, bn_ref, o_ref):
        h0 = jnp.maximum(_dot(x_ref[...], we_ref[...]) + be_ref[...], 0.0)
        o_ref[...] = _dot(h0, wn_ref[...]) + bn_ref[...]

    blk = 400
    return pl.pallas_call(
        body,
        grid=(_N // blk,),
        in_specs=[
            pl.BlockSpec((blk, _D), lambda i: (i, 0)),
            pl.BlockSpec((_D, _D), lambda i: (0, 0)),
            pl.BlockSpec((1, _D), lambda i: (0, 0)),
            pl.BlockSpec((_D, _D), lambda i: (0, 0)),
            pl.BlockSpec((1, _D), lambda i: (0, 0)),
        ],
        out_specs=pl.BlockSpec((blk, _D), lambda i: (i, 0)),
        out_shape=jax.ShapeDtypeStruct((_N, _D), jnp.float32),
    )(x, enc_w, enc_b, node_w, node_b)


def _tc_post(xh, seg, eac, wt2, m2aug, upd_top, upd_b, ln_g, ln_b,
             next_w=None, next_b=None):
    last = next_w is None

    def body(*refs):
        if last:
            (xh_ref, s0_ref, s1_ref, ec0_ref, ec1_ref,
             wt2_ref, m2_ref, ut_ref, ub_ref, g_ref, b_ref,
             o_ref, hs_ref) = refs
        else:
            (xh_ref, s0_ref, s1_ref, ec0_ref, ec1_ref,
             wt2_ref, m2_ref, ut_ref, ub_ref, g_ref, b_ref,
             nw_ref, nb_ref, o_ref) = refs
        seg_b = s0_ref[...] + s1_ref[...]
        eac_b = ec0_ref[...] + ec1_ref[...]
        cnt = eac_b[:, 16:17]
        pre = _dot(seg_b, wt2_ref[...]) + _dot(eac_b, m2_ref[...])
        inv = 1.0 / jnp.maximum(cnt, 1.0)
        out = _dot(xh_ref[...], ut_ref[...]) + pre * inv + ub_ref[...]
        mu = jnp.mean(out, axis=-1, keepdims=True)
        var = jnp.mean((out - mu) ** 2, axis=-1, keepdims=True)
        out = (out - mu) * lax.rsqrt(var + 1e-5) * g_ref[...] + b_ref[...]
        h = jnp.maximum(out, 0.0)
        if last:
            o_ref[...] = h
            i = pl.program_id(0)

            @pl.when(i == 0)
            def _():
                hs_ref[...] = jnp.zeros_like(hs_ref)

            hs_ref[...] += jnp.sum(h, axis=0, keepdims=True)
        else:
            o_ref[...] = _dot(h, nw_ref[...]) + nb_ref[...]

    blk = 400
    nb = pl.BlockSpec((blk, _D), lambda i: (i, 0))
    cst = lambda r, c: pl.BlockSpec((r, c), lambda i: (0, 0))
    in_specs = [nb, nb, nb, nb, nb,
                cst(_D, _D), cst(_D, _D),
                cst(_D, _D), cst(1, _D), cst(1, _D), cst(1, _D)]
    args = [xh, seg[0], seg[1], eac[0], eac[1],
            wt2, m2aug, upd_top, upd_b, ln_g, ln_b]
    if last:
        out_specs = [nb, pl.BlockSpec((1, _D), lambda i: (0, 0))]
        out_shape = [jax.ShapeDtypeStruct((_N, _D), jnp.float32),
                     jax.ShapeDtypeStruct((1, _D), jnp.float32)]
    else:
        in_specs += [cst(_D, _D), cst(1, _D)]
        args += [next_w, next_b]
        out_specs = nb
        out_shape = jax.ShapeDtypeStruct((_N, _D), jnp.float32)
    return pl.pallas_call(
        body,
        grid=(_N // blk,),
        in_specs=in_specs,
        out_specs=out_specs,
        out_shape=out_shape,
    )(*args)


def _tc_graph(hsum, pool_w, pool_b):
    def body(hs_ref, w_ref, b_ref, o_ref):
        mean = hs_ref[...] * (1.0 / _N)
        o_ref[...] = jnp.maximum(_dot(mean, w_ref[...]) + b_ref[...], 0.0)

    return pl.pallas_call(
        body,
        grid=(1,),
        in_specs=[
            pl.BlockSpec((1, _D), lambda i: (0, 0)),
            pl.BlockSpec((_D, _D), lambda i: (0, 0)),
            pl.BlockSpec((1, _D), lambda i: (0, 0)),
        ],
        out_specs=pl.BlockSpec((1, _D), lambda i: (0, 0)),
        out_shape=jax.ShapeDtypeStruct((1, _D), jnp.float32),
    )(hsum, pool_w, pool_b)


@jax.jit
def kernel(x, edge_index, edge_attr, params):
    ei1 = edge_index.reshape(2 * _E)

    w_bd = jnp.kron(jnp.eye(8, dtype=jnp.float32), params['enc_e_W'])
    b_tile = jnp.tile(params['enc_e_b'], 8)[None, :]
    lw = []
    for lp in params['layers']:
        wt = lp['msg_W'][:_D]
        wb = lp['msg_W'][_D:]
        m = lp['edge_W'] @ wb
        cvec = lp['edge_b'] @ wb + lp['msg_b']
        upd_top = lp['upd_W'][:_D]
        upd_bot = lp['upd_W'][_D:]
        m2 = m @ upd_bot
        c2 = (cvec @ upd_bot)[None, :]
        m2aug = jnp.concatenate(
            [m2, c2, jnp.zeros((_D - _ED - 1, _D), jnp.float32)], axis=0)
        lw.append({
            'wt2': wt @ upd_bot,
            'm2aug': m2aug,
            'upd_top': upd_top,
            'upd_b': lp['upd_b'][None, :],
            'ln_g': lp['ln_g'][None, :],
            'ln_b': lp['ln_b'][None, :],
            'node_w': lp['node_W'],
            'node_b': lp['node_b'][None, :],
        })

    ea_packed = _tc_ea(edge_attr.reshape(_E // 8, 128), w_bd, b_tile)
    ea = ea_packed.reshape(_E, _ED)

    eac = _sc_eac(ea_packed, ei1)

    xh = _tc_head(x, params['enc_n_W'], params['enc_n_b'][None, :],
                  lw[0]['node_w'], lw[0]['node_b'])

    seg1 = _sc_seg(xh, ei1)
    xh2 = _tc_post(xh, seg1, eac, lw[0]['wt2'], lw[0]['m2aug'],
                   lw[0]['upd_top'], lw[0]['upd_b'],
                   lw[0]['ln_g'], lw[0]['ln_b'],
                   next_w=lw[1]['node_w'], next_b=lw[1]['node_b'])

    seg2 = _sc_seg(xh2, ei1)
    h, hsum = _tc_post(xh2, seg2, eac, lw[1]['wt2'], lw[1]['m2aug'],
                       lw[1]['upd_top'], lw[1]['upd_b'],
                       lw[1]['ln_g'], lw[1]['ln_b'])

    graph = _tc_graph(hsum, params['pool_W'], params['pool_b'][None, :])
    return (h, ea, graph)


# speedup vs baseline: 1.5053x; 1.0790x over previous
import functools
import jax
import jax.numpy as jnp
from jax import lax
from jax.experimental import pallas as pl
from jax.experimental.pallas import tpu as pltpu
from jax.experimental.pallas import tpu_sc as plsc

_N = 10000
_E = 320000
_D = 128
_ED = 16

_NW = 32
_EPW = _E // _NW
_C = 80
_NCHUNK = _EPW // _C
_NT = 10240
_STRIPE = _NT // 16
_BB = 32


def _zero_vmem_2d(ref, rows, cols):
    per_row = cols // 16
    zeros16 = jnp.zeros((16,), jnp.float32)

    def body(t, carry):
        r = t // per_row
        c = (t % per_row) * 16
        ref[r, pl.ds(c, 16)] = zeros16
        return carry

    lax.fori_loop(0, rows * per_row, body, 0)


def _sc_seg(rows, src1, dst1):
    mesh = plsc.VectorSubcoreMesh(core_axis_name="c", subcore_axis_name="s")
    NS = 4

    @functools.partial(
        pl.kernel,
        mesh=mesh,
        out_type=jax.ShapeDtypeStruct((2, _NT, _D), jnp.float32),
        scratch_types=(
            [pltpu.VMEM((_C,), jnp.int32)] * NS
            + [pltpu.VMEM((_C,), jnp.int32)] * NS
            + [pltpu.VMEM((_C, _D), jnp.float32)] * NS
            + [pltpu.VMEM((_BB, _D), jnp.float32),
               pltpu.VMEM_SHARED((_NT, _D), jnp.float32)]
            + [pltpu.SemaphoreType.DMA] * (3 * NS)
        ),
    )
    def k(rows_hbm, src_hbm, dst_hbm, out_hbm, *rest):
        sbufs = rest[0:NS]
        dbufs = rest[NS:2 * NS]
        rbufs = rest[2 * NS:3 * NS]
        bounce_v = rest[3 * NS]
        table_sh = rest[3 * NS + 1]
        gsems = rest[3 * NS + 2:3 * NS + 2 + NS]
        ssems = rest[3 * NS + 2 + NS:3 * NS + 2 + 2 * NS]
        dsems = rest[3 * NS + 2 + 2 * NS:3 * NS + 2 + 3 * NS]

        c = lax.axis_index("c")
        s = lax.axis_index("s")
        wid = s * 2 + c

        def sidx_load(j, b):
            pltpu.async_copy(src_hbm.at[pl.ds(wid * _EPW + j * _C, _C)],
                             sbufs[b], ssems[b])

        def sidx_wait(j, b):
            pltpu.make_async_copy(src_hbm.at[pl.ds(wid * _EPW + j * _C, _C)],
                                  sbufs[b], ssems[b]).wait()

        def didx_load(j, b):
            pltpu.async_copy(dst_hbm.at[pl.ds(wid * _EPW + j * _C, _C)],
                             dbufs[b], dsems[b])

        def didx_wait(j, b):
            pltpu.make_async_copy(
                dst_hbm.at[pl.ds(wid * _EPW + j * _C, _C)],
                dbufs[b], dsems[b]).wait()

        def gather(j, b):
            pltpu.async_copy(rows_hbm.at[sbufs[b]], rbufs[b], gsems[b])

        def gwait(j, b):
            pltpu.make_async_copy(rows_hbm.at[sbufs[b]], rbufs[b],
                                  gsems[b]).wait()

        def scatter(j, b):
            pltpu.sync_copy(rbufs[b], table_sh.at[dbufs[b]], add=True)

        for j in range(NS):
            sidx_load(j, j)
            didx_load(j, j)

        _zero_vmem_2d(bounce_v, _BB, _D)

        def zbody(j, carry):
            pltpu.sync_copy(bounce_v,
                            table_sh.at[pl.ds(s * _STRIPE + j * _BB, _BB)])
            return carry

        lax.fori_loop(0, _STRIPE // _BB, zbody, 0)
        for j in range(2):
            sidx_wait(j, j)
        plsc.subcore_barrier()

        gather(0, 0)
        gather(1, 1)

        def body(g, carry):
            for sub in range(NS):
                j = g * NS + sub
                bj = sub

                b2 = (sub + 2) % NS

                @pl.when(j + 2 < _NCHUNK)
                def _():
                    sidx_wait(j + 2, b2)
                    gather(j + 2, b2)

                @pl.when(j < _NCHUNK)
                def _():
                    gwait(j, bj)
                    didx_wait(j, bj)

                    @pl.when(j + NS < _NCHUNK)
                    def _():
                        sidx_load(j + NS, bj)

                    scatter(j, bj)

                    @pl.when(j + NS < _NCHUNK)
                    def _():
                        didx_load(j + NS, bj)

            return carry

        lax.fori_loop(0, (_NCHUNK + NS - 1) // NS, body, 0)
        plsc.subcore_barrier()

        def rbody(j, carry):
            r0 = s * _STRIPE + j * _BB
            pltpu.sync_copy(table_sh.at[pl.ds(r0, _BB)], bounce_v)
            pltpu.sync_copy(bounce_v, out_hbm.at[c, pl.ds(r0, _BB)])
            return carry

        lax.fori_loop(0, _STRIPE // _BB, rbody, 0)

    return k(rows, src1, dst1)


def _sc_eac(ea_packed, dst1):
    mesh = plsc.VectorSubcoreMesh(core_axis_name="c", subcore_axis_name="s")
    CE = 64
    TOTC = _E // CE
    KMAX = (TOTC + _NW - 1) // _NW

    @functools.partial(
        pl.kernel,
        mesh=mesh,
        out_type=jax.ShapeDtypeStruct((2, _NT, _D), jnp.float32),
        scratch_types=[
            pltpu.VMEM((CE,), jnp.int32),
            pltpu.VMEM((CE,), jnp.int32),
            pltpu.VMEM((CE // 8, _D), jnp.float32),
            pltpu.VMEM((CE // 8, _D), jnp.float32),
            pltpu.VMEM((CE, _D), jnp.float32),
            pltpu.VMEM((_BB, _D), jnp.float32),
            pltpu.VMEM_SHARED((_NT, _D), jnp.float32),
            pltpu.SemaphoreType.DMA,
            pltpu.SemaphoreType.DMA,
            pltpu.SemaphoreType.DMA,
            pltpu.SemaphoreType.DMA,
        ],
    )
    def k(eap_hbm, dst_hbm, out_hbm, db0_v, db1_v, p0_v, p1_v, rows_v,
          bounce_v, table_sh, p0_sem, p1_sem, d0_sem, d1_sem):
        c = lax.axis_index("c")
        s = lax.axis_index("s")
        wid = s * 2 + c

        dbufs = (db0_v, db1_v)
        pbufs = (p0_v, p1_v)
        psems = (p0_sem, p1_sem)
        dsems = (d0_sem, d1_sem)

        def ch(kk):
            return wid + kk * _NW

        def didx_load(kk, b):
            pltpu.async_copy(dst_hbm.at[pl.ds(ch(kk) * CE, CE)],
                             dbufs[b], dsems[b])

        def didx_wait(kk, b):
            pltpu.make_async_copy(dst_hbm.at[pl.ds(ch(kk) * CE, CE)],
                                  dbufs[b], dsems[b]).wait()

        def prow(kk):
            return pl.multiple_of(ch(kk) * (CE // 8), 8)

        def pload(kk, b):
            pltpu.async_copy(eap_hbm.at[pl.ds(prow(kk), CE // 8)],
                             pbufs[b], psems[b])

        def pwait(kk, b):
            pltpu.make_async_copy(eap_hbm.at[pl.ds(prow(kk), CE // 8)],
                                  pbufs[b], psems[b]).wait()

        didx_load(0, 0)
        pload(0, 0)
        didx_load(1, 1)
        pload(1, 1)

        _zero_vmem_2d(rows_v, CE, _D)
        onehot = jnp.where(lax.iota(jnp.int32, 16) == 0, 1.0, 0.0)
        for e in range(CE):
            rows_v[e, pl.ds(16, 16)] = onehot

        _zero_vmem_2d(bounce_v, _BB, _D)

        def zbody(j, carry):
            pltpu.sync_copy(bounce_v,
                            table_sh.at[pl.ds(s * _STRIPE + j * _BB, _BB)])
            return carry

        lax.fori_loop(0, _STRIPE // _BB, zbody, 0)
        plsc.subcore_barrier()

        def process(kk, b):
            pwait(kk, b)
            pbuf = pbufs[b]
            for e in range(CE):
                rows_v[e, pl.ds(0, 16)] = pbuf[e // 8,
                                               pl.ds((e % 8) * 16, 16)]
            didx_wait(kk, b)
            pltpu.sync_copy(rows_v, table_sh.at[dbufs[b]], add=True)

            @pl.when(ch(kk + 2) < TOTC)
            def _():
                didx_load(kk + 2, b)
                pload(kk + 2, b)

        def body(g, carry):
            k0 = g * 2

            @pl.when(ch(k0) < TOTC)
            def _():
                process(k0, 0)

            @pl.when(ch(k0 + 1) < TOTC)
            def _():
                process(k0 + 1, 1)

            return carry

        lax.fori_loop(0, (KMAX + 1) // 2, body, 0)
        plsc.subcore_barrier()

        def rbody(j, carry):
            r0 = s * _STRIPE + j * _BB
            pltpu.sync_copy(table_sh.at[pl.ds(r0, _BB)], bounce_v)
            pltpu.sync_copy(bounce_v, out_hbm.at[c, pl.ds(r0, _BB)])
            return carry

        lax.fori_loop(0, _STRIPE // _BB, rbody, 0)

    return k(ea_packed, dst1)


def _dot(a, b):
    return jnp.dot(a, b, preferred_element_type=jnp.float32)


def _tc_ea(ea_packed, w_bd, b_tile):
    def body(x_ref, w_ref, b_ref, o_ref):
        o_ref[...] = jnp.maximum(_dot(x_ref[...], w_ref[...]) + b_ref[...], 0.0)

    nblk = 40
    return pl.pallas_call(
        body,
        grid=(nblk,),
        in_specs=[
            pl.BlockSpec((_E // 8 // nblk, 128), lambda i: (i, 0)),
            pl.BlockSpec((128, 128), lambda i: (0, 0)),
            pl.BlockSpec((1, 128), lambda i: (0, 0)),
        ],
        out_specs=pl.BlockSpec((_E // 8 // nblk, 128), lambda i: (i, 0)),
        out_shape=jax.ShapeDtypeStruct((_E // 8, 128), jnp.float32),
    )(ea_packed, w_bd, b_tile)


def _tc_ei(edge_index):
    def body(x_ref, os_ref, od_ref):
        os_ref[...] = x_ref[0, :]
        od_ref[...] = x_ref[1, :]

    return pl.pallas_call(
        body,
        grid=(1,),
        in_specs=[pl.BlockSpec((2, _E), lambda i: (0, 0))],
        out_specs=[pl.BlockSpec((_E,), lambda i: (0,)),
                   pl.BlockSpec((_E,), lambda i: (0,))],
        out_shape=[jax.ShapeDtypeStruct((_E,), jnp.int32),
                   jax.ShapeDtypeStruct((_E,), jnp.int32)],
    )(edge_index)


def _tc_head(x, enc_w, enc_b, node_w, node_b):
    def body(x_ref, we_ref, be_ref, wn_ref, bn_ref, o_ref):
        h0 = jnp.maximum(_dot(x_ref[...], we_ref[...]) + be_ref[...], 0.0)
        o_ref[...] = _dot(h0, wn_ref[...]) + bn_ref[...]

    blk = 400
    return pl.pallas_call(
        body,
        grid=(_N // blk,),
        in_specs=[
            pl.BlockSpec((blk, _D), lambda i: (i, 0)),
            pl.BlockSpec((_D, _D), lambda i: (0, 0)),
            pl.BlockSpec((1, _D), lambda i: (0, 0)),
            pl.BlockSpec((_D, _D), lambda i: (0, 0)),
            pl.BlockSpec((1, _D), lambda i: (0, 0)),
        ],
        out_specs=pl.BlockSpec((blk, _D), lambda i: (i, 0)),
        out_shape=jax.ShapeDtypeStruct((_N, _D), jnp.float32),
    )(x, enc_w, enc_b, node_w, node_b)


def _tc_post(xh, seg, eac, wt2, m2aug, upd_top, upd_b, ln_g, ln_b,
             next_w=None, next_b=None):
    last = next_w is None

    def body(*refs):
        if last:
            (xh_ref, s0_ref, s1_ref, ec0_ref, ec1_ref,
             wt2_ref, m2_ref, ut_ref, ub_ref, g_ref, b_ref,
             o_ref, hs_ref) = refs
        else:
            (xh_ref, s0_ref, s1_ref, ec0_ref, ec1_ref,
             wt2_ref, m2_ref, ut_ref, ub_ref, g_ref, b_ref,
             nw_ref, nb_ref, o_ref) = refs
        seg_b = s0_ref[...] + s1_ref[...]
        eac_b = ec0_ref[...] + ec1_ref[...]
        cnt = eac_b[:, 16:17]
        pre = _dot(seg_b, wt2_ref[...]) + _dot(eac_b, m2_ref[...])
        inv = 1.0 / jnp.maximum(cnt, 1.0)
        out = _dot(xh_ref[...], ut_ref[...]) + pre * inv + ub_ref[...]
        mu = jnp.mean(out, axis=-1, keepdims=True)
        var = jnp.mean((out - mu) ** 2, axis=-1, keepdims=True)
        out = (out - mu) * lax.rsqrt(var + 1e-5) * g_ref[...] + b_ref[...]
        h = jnp.maximum(out, 0.0)
        if last:
            o_ref[...] = h
            i = pl.program_id(0)

            @pl.when(i == 0)
            def _():
                hs_ref[...] = jnp.zeros_like(hs_ref)

            hs_ref[...] += jnp.sum(h, axis=0, keepdims=True)
        else:
            o_ref[...] = _dot(h, nw_ref[...]) + nb_ref[...]

    blk = 400
    nb = pl.BlockSpec((blk, _D), lambda i: (i, 0))
    cst = lambda r, c: pl.BlockSpec((r, c), lambda i: (0, 0))
    in_specs = [nb, nb, nb, nb, nb,
                cst(_D, _D), cst(_D, _D),
                cst(_D, _D), cst(1, _D), cst(1, _D), cst(1, _D)]
    args = [xh, seg[0], seg[1], eac[0], eac[1],
            wt2, m2aug, upd_top, upd_b, ln_g, ln_b]
    if last:
        out_specs = [nb, pl.BlockSpec((1, _D), lambda i: (0, 0))]
        out_shape = [jax.ShapeDtypeStruct((_N, _D), jnp.float32),
                     jax.ShapeDtypeStruct((1, _D), jnp.float32)]
    else:
        in_specs += [cst(_D, _D), cst(1, _D)]
        args += [next_w, next_b]
        out_specs = nb
        out_shape = jax.ShapeDtypeStruct((_N, _D), jnp.float32)
    return pl.pallas_call(
        body,
        grid=(_N // blk,),
        in_specs=in_specs,
        out_specs=out_specs,
        out_shape=out_shape,
    )(*args)


def _tc_graph(hsum, pool_w, pool_b):
    def body(hs_ref, w_ref, b_ref, o_ref):
        mean = hs_ref[...] * (1.0 / _N)
        o_ref[...] = jnp.maximum(_dot(mean, w_ref[...]) + b_ref[...], 0.0)

    return pl.pallas_call(
        body,
        grid=(1,),
        in_specs=[
            pl.BlockSpec((1, _D), lambda i: (0, 0)),
            pl.BlockSpec((_D, _D), lambda i: (0, 0)),
            pl.BlockSpec((1, _D), lambda i: (0, 0)),
        ],
        out_specs=pl.BlockSpec((1, _D), lambda i: (0, 0)),
        out_shape=jax.ShapeDtypeStruct((1, _D), jnp.float32),
    )(hsum, pool_w, pool_b)


@jax.jit
def kernel(x, edge_index, edge_attr, params):
    src1, dst1 = _tc_ei(edge_index)

    w_bd = jnp.kron(jnp.eye(8, dtype=jnp.float32), params['enc_e_W'])
    b_tile = jnp.tile(params['enc_e_b'], 8)[None, :]
    lw = []
    for lp in params['layers']:
        wt = lp['msg_W'][:_D]
        wb = lp['msg_W'][_D:]
        m = lp['edge_W'] @ wb
        cvec = lp['edge_b'] @ wb + lp['msg_b']
        upd_top = lp['upd_W'][:_D]
        upd_bot = lp['upd_W'][_D:]
        m2 = m @ upd_bot
        c2 = (cvec @ upd_bot)[None, :]
        m2aug = jnp.concatenate(
            [m2, c2, jnp.zeros((_D - _ED - 1, _D), jnp.float32)], axis=0)
        lw.append({
            'wt2': wt @ upd_bot,
            'm2aug': m2aug,
            'upd_top': upd_top,
            'upd_b': lp['upd_b'][None, :],
            'ln_g': lp['ln_g'][None, :],
            'ln_b': lp['ln_b'][None, :],
            'node_w': lp['node_W'],
            'node_b': lp['node_b'][None, :],
        })

    ea_packed = _tc_ea(edge_attr.reshape(_E // 8, 128), w_bd, b_tile)
    ea = ea_packed.reshape(_E, _ED)

    eac = _sc_eac(ea_packed, dst1)

    xh = _tc_head(x, params['enc_n_W'], params['enc_n_b'][None, :],
                  lw[0]['node_w'], lw[0]['node_b'])

    seg1 = _sc_seg(xh, src1, dst1)
    xh2 = _tc_post(xh, seg1, eac, lw[0]['wt2'], lw[0]['m2aug'],
                   lw[0]['upd_top'], lw[0]['upd_b'],
                   lw[0]['ln_g'], lw[0]['ln_b'],
                   next_w=lw[1]['node_w'], next_b=lw[1]['node_b'])

    seg2 = _sc_seg(xh2, src1, dst1)
    h, hsum = _tc_post(xh2, seg2, eac, lw[1]['wt2'], lw[1]['m2aug'],
                       lw[1]['upd_top'], lw[1]['upd_b'],
                       lw[1]['ln_g'], lw[1]['ln_b'])

    graph = _tc_graph(hsum, params['pool_W'], params['pool_b'][None, :])
    return (h, ea, graph)
